# baseline (speedup 1.0000x reference)
import functools

import jax
import jax.numpy as jnp
from jax import lax
from jax.experimental import pallas as pl
from jax.experimental.pallas import tpu as pltpu
from jax.experimental.pallas import tpu_sc as plsc

N = 10000
E = 320000
D = 128
DH = 64

NSUB = 16
CHUNK = 128
NBUF = 2
CHUNKS = 160
EPAD = NSUB * CHUNKS * CHUNK
EROWS = EPAD // 128
EVALID_ROWS = E // 128

NOUT = 10240
NACC = 10368
DUMMY = NOUT
SLAB0 = NACC // NSUB
SLAB1 = NOUT // NSUB
DEGW = 8

_sc_mesh = plsc.VectorSubcoreMesh(core_axis_name="c", subcore_axis_name="s")


def _pre_body(e_ref, x_ref, w1o_ref, w1r_ref, colp_ref, ys_ref, r1_ref):
    row = e_ref[0]
    col = e_ref[1]
    ridx = lax.broadcasted_iota(jnp.int32, (EROWS, 128), 0)
    valid = (ridx < EVALID_ROWS) & (row != col)
    colp_ref[...] = jnp.where(valid, col, DUMMY)
    x = x_ref[...]
    dn = (((1,), (1,)), ((), ()))
    y1 = lax.dot_general(x, w1o_ref[...], dn,
                         preferred_element_type=jnp.float32)
    ys_ref[0] = y1[:, :DH]
    ys_ref[1] = y1[:, DH:]
    r1_ref[...] = lax.dot_general(x, w1r_ref[...], dn,
                                  preferred_element_type=jnp.float32)


_pre = pl.pallas_call(
    _pre_body,
    out_shape=(
        jax.ShapeDtypeStruct((EROWS, 128), jnp.int32),
        jax.ShapeDtypeStruct((2, N, DH), jnp.float32),
        jax.ShapeDtypeStruct((N, D), jnp.float32),
    ),
)


def _segsum_body(ys, rowp, colp, zslab, dslab, ones, zout, dout,
                 rowv, colv, rows, onesv, zacc, dacc, gsem, ssem, dsem):
    c = lax.axis_index("c")
    s = lax.axis_index("s")

    pltpu.sync_copy(zslab, zacc.at[pl.ds(s * SLAB0, SLAB0)])
    pltpu.sync_copy(dslab, dacc.at[pl.ds(s * SLAB0, SLAB0)])
    pltpu.sync_copy(rowp.at[s], rowv)
    pltpu.sync_copy(colp.at[s], colv)
    pltpu.sync_copy(ones, onesv)
    plsc.subcore_barrier()

    yhalf = ys.at[c]

    def gather(j, b):
        pltpu.async_copy(yhalf.at[rowv.at[j]], rows.at[b], gsem.at[b])

    def gather_wait(j, b):
        pltpu.make_async_copy(yhalf.at[rowv.at[j]], rows.at[b],
                              gsem.at[b]).wait()

    for b in range(NBUF):
        gather(b, b)

    def body(i, carry):
        j = i * NBUF
        for b in range(NBUF):
            gather_wait(j + b, b)
            pltpu.sync_copy(rows.at[b], zacc.at[colv.at[j + b]], add=True)

            @pl.when(c == 0)
            def _():
                pltpu.sync_copy(onesv, dacc.at[colv.at[j + b]], add=True)

            @pl.when(j + b + NBUF < CHUNKS)
            def _():
                gather(j + b + NBUF, b)

        return carry

    lax.fori_loop(0, CHUNKS // NBUF, body, 0)
    plsc.subcore_barrier()

    pltpu.sync_copy(zacc.at[pl.ds(s * SLAB1, SLAB1)],
                    zout.at[c, pl.ds(s * SLAB1, SLAB1)])

    @pl.when(c == 0)
    def _():
        pltpu.sync_copy(dacc.at[pl.ds(s * SLAB1, SLAB1)],
                        dout.at[pl.ds(s * SLAB1, SLAB1)])


_segsum = functools.partial(
    pl.kernel,
    out_type=(
        jax.ShapeDtypeStruct((2, NOUT, DH), jnp.float32),
        jax.ShapeDtypeStruct((NOUT, DEGW), jnp.float32),
    ),
    mesh=_sc_mesh,
    scratch_types=[
        pltpu.VMEM((CHUNKS, CHUNK), jnp.int32),
        pltpu.VMEM((CHUNKS, CHUNK), jnp.int32),
        pltpu.VMEM((NBUF, CHUNK, DH), jnp.float32),
        pltpu.VMEM((CHUNK, DEGW), jnp.float32),
        pltpu.VMEM_SHARED((NACC, DH), jnp.float32),
        pltpu.VMEM_SHARED((NACC, DEGW), jnp.float32),
        pltpu.SemaphoreType.DMA((NBUF,)),
        pltpu.SemaphoreType.DMA((NBUF,)),
        pltpu.SemaphoreType.DMA,
    ],
    compiler_params=pltpu.CompilerParams(use_tc_tiling_on_sc=False),
)(_segsum_body)


def _combine1_body(zp, dp, ys, r1, b1, w2o, w2r, y2s_ref, r2_ref):
    z = jnp.concatenate([zp[0, :N, :], zp[1, :N, :]], axis=1)
    y1 = jnp.concatenate([ys[0], ys[1]], axis=1)
    deg = dp[:N, 0:1]
    deginv = 1.0 / (deg + 1.0)
    h = jnp.maximum((z + y1) * deginv + b1[...] + r1[...], 0.0)
    dn = (((1,), (1,)), ((), ()))
    y2 = lax.dot_general(h, w2o[...], dn,
                         preferred_element_type=jnp.float32)
    y2s_ref[0] = y2[:, :DH]
    y2s_ref[1] = y2[:, DH:]
    r2_ref[...] = lax.dot_general(h, w2r[...], dn,
                                  preferred_element_type=jnp.float32)


_combine1 = pl.pallas_call(
    _combine1_body,
    out_shape=(
        jax.ShapeDtypeStruct((2, N, DH), jnp.float32),
        jax.ShapeDtypeStruct((N, D), jnp.float32),
    ),
)


def _combine2_body(zp, dp, y2s, r2, b2, out_ref):
    z = jnp.concatenate([zp[0, :N, :], zp[1, :N, :]], axis=1)
    y2 = jnp.concatenate([y2s[0], y2s[1]], axis=1)
    deg = dp[:N, 0:1]
    deginv = 1.0 / (deg + 1.0)
    out_ref[...] = (z + y2) * deginv + b2[...] + r2[...]


_combine2 = pl.pallas_call(
    _combine2_body,
    out_shape=jax.ShapeDtypeStruct((N, D), jnp.float32),
)


def kernel(x, edge_index, W1_out, b1, W1_root, W2_out, b2, W2_root):
    e = edge_index.astype(jnp.int32)
    e = jnp.pad(e, ((0, 0), (0, EPAD - E)))
    e2 = e.reshape(2, EROWS, 128)

    colp, y1s, r1 = _pre(e2, x, W1_out, W1_root)

    rowp3 = e2[0].reshape(NSUB, CHUNKS, CHUNK)
    colp3 = colp.reshape(NSUB, CHUNKS, CHUNK)

    zslab = jnp.zeros((SLAB0, DH), jnp.float32)
    dslab = jnp.zeros((SLAB0, DEGW), jnp.float32)
    ones = jnp.ones((CHUNK, DEGW), jnp.float32)

    zp1, dp = _segsum(y1s, rowp3, colp3, zslab, dslab, ones)
    y2s, r2 = _combine1(zp1, dp, y1s, r1, b1.reshape(1, D), W2_out, W2_root)
    zp2, _ = _segsum(y2s, rowp3, colp3, zslab, dslab, ones)
    return _combine2(zp2, dp, y2s, r2, b2.reshape(1, D))


# speedup vs baseline: 1.0098x; 1.0098x over previous
import functools

import jax
import jax.numpy as jnp
from jax import lax
from jax.experimental import pallas as pl
from jax.experimental.pallas import tpu as pltpu
from jax.experimental.pallas import tpu_sc as plsc

N = 10000
E = 320000
D = 128
DH = 64

NSUB = 16
CHUNK = 128
NBUF = 2
CHUNKS = 160
EPAD = NSUB * CHUNKS * CHUNK
EROWS = EPAD // 128
EVALID_ROWS = E // 128

NOUT = 10240
NACC = 10368
DUMMY = NOUT
SLAB0 = NACC // NSUB
SLAB1 = NOUT // NSUB
DEGW = 8

_sc_mesh = plsc.VectorSubcoreMesh(core_axis_name="c", subcore_axis_name="s")


def _pre_body(e_ref, x_ref, w1o_ref, w1r_ref, colp_ref, ys_ref, r1_ref):
    row = e_ref[0]
    col = e_ref[1]
    ridx = lax.broadcasted_iota(jnp.int32, (EROWS, 128), 0)
    valid = (ridx < EVALID_ROWS) & (row != col)
    colp_ref[...] = jnp.where(valid, col, DUMMY)
    x = x_ref[...]
    dn = (((1,), (1,)), ((), ()))
    y1 = lax.dot_general(x, w1o_ref[...], dn,
                         preferred_element_type=jnp.float32)
    ys_ref[0] = y1[:, :DH]
    ys_ref[1] = y1[:, DH:]
    r1_ref[...] = lax.dot_general(x, w1r_ref[...], dn,
                                  preferred_element_type=jnp.float32)


_pre = pl.pallas_call(
    _pre_body,
    out_shape=(
        jax.ShapeDtypeStruct((EROWS, 128), jnp.int32),
        jax.ShapeDtypeStruct((2, N, DH), jnp.float32),
        jax.ShapeDtypeStruct((N, D), jnp.float32),
    ),
)


def _segsum_body(ys, rowp, colp, zslab, dslab, ones, zout, dout,
                 rowv, colv, rows0, rows1, onesv, zacc, dacc, sem0, sem1):
    c = lax.axis_index("c")
    s = lax.axis_index("s")

    pltpu.sync_copy(zslab, zacc.at[pl.ds(s * SLAB0, SLAB0)])
    pltpu.sync_copy(dslab, dacc.at[pl.ds(s * SLAB0, SLAB0)])
    pltpu.sync_copy(rowp.at[s], rowv)
    pltpu.sync_copy(colp.at[s], colv)
    pltpu.sync_copy(ones, onesv)
    plsc.subcore_barrier()

    yhalf = ys.at[c]

    def gather(j, buf, sem):
        pltpu.async_copy(yhalf.at[rowv.at[j]], buf, sem)

    def gather_wait(j, buf, sem):
        pltpu.make_async_copy(yhalf.at[rowv.at[j]], buf, sem).wait()

    def scatter(j, buf):
        pltpu.sync_copy(buf, zacc.at[colv.at[j]], add=True)

        @pl.when(c == 0)
        def _():
            pltpu.sync_copy(onesv, dacc.at[colv.at[j]], add=True)

    gather(0, rows0, sem0)
    gather(1, rows1, sem1)

    def body(i, carry):
        j = 2 * i
        gather_wait(j, rows0, sem0)
        scatter(j, rows0)

        @pl.when(j + 2 < CHUNKS)
        def _():
            gather(j + 2, rows0, sem0)

        gather_wait(j + 1, rows1, sem1)
        scatter(j + 1, rows1)

        @pl.when(j + 3 < CHUNKS)
        def _():
            gather(j + 3, rows1, sem1)

        return carry

    lax.fori_loop(0, CHUNKS // 2, body, 0)
    plsc.subcore_barrier()

    pltpu.sync_copy(zacc.at[pl.ds(s * SLAB1, SLAB1)],
                    zout.at[c, pl.ds(s * SLAB1, SLAB1)])

    @pl.when(c == 0)
    def _():
        pltpu.sync_copy(dacc.at[pl.ds(s * SLAB1, SLAB1)],
                        dout.at[pl.ds(s * SLAB1, SLAB1)])


_segsum = functools.partial(
    pl.kernel,
    out_type=(
        jax.ShapeDtypeStruct((2, NOUT, DH), jnp.float32),
        jax.ShapeDtypeStruct((NOUT, DEGW), jnp.float32),
    ),
    mesh=_sc_mesh,
    scratch_types=[
        pltpu.VMEM((CHUNKS, CHUNK), jnp.int32),
        pltpu.VMEM((CHUNKS, CHUNK), jnp.int32),
        pltpu.VMEM((CHUNK, DH), jnp.float32),
        pltpu.VMEM((CHUNK, DH), jnp.float32),
        pltpu.VMEM((CHUNK, DEGW), jnp.float32),
        pltpu.VMEM_SHARED((NACC, DH), jnp.float32),
        pltpu.VMEM_SHARED((NACC, DEGW), jnp.float32),
        pltpu.SemaphoreType.DMA,
        pltpu.SemaphoreType.DMA,
    ],
    compiler_params=pltpu.CompilerParams(use_tc_tiling_on_sc=False),
)(_segsum_body)


def _combine1_body(zp, dp, ys, r1, b1, w2o, w2r, y2s_ref, r2_ref):
    z = jnp.concatenate([zp[0, :N, :], zp[1, :N, :]], axis=1)
    y1 = jnp.concatenate([ys[0], ys[1]], axis=1)
    deg = dp[:N, 0:1]
    deginv = 1.0 / (deg + 1.0)
    h = jnp.maximum((z + y1) * deginv + b1[...] + r1[...], 0.0)
    dn = (((1,), (1,)), ((), ()))
    y2 = lax.dot_general(h, w2o[...], dn,
                         preferred_element_type=jnp.float32)
    y2s_ref[0] = y2[:, :DH]
    y2s_ref[1] = y2[:, DH:]
    r2_ref[...] = lax.dot_general(h, w2r[...], dn,
                                  preferred_element_type=jnp.float32)


_combine1 = pl.pallas_call(
    _combine1_body,
    out_shape=(
        jax.ShapeDtypeStruct((2, N, DH), jnp.float32),
        jax.ShapeDtypeStruct((N, D), jnp.float32),
    ),
)


def _combine2_body(zp, dp, y2s, r2, b2, out_ref):
    z = jnp.concatenate([zp[0, :N, :], zp[1, :N, :]], axis=1)
    y2 = jnp.concatenate([y2s[0], y2s[1]], axis=1)
    deg = dp[:N, 0:1]
    deginv = 1.0 / (deg + 1.0)
    out_ref[...] = (z + y2) * deginv + b2[...] + r2[...]


_combine2 = pl.pallas_call(
    _combine2_body,
    out_shape=jax.ShapeDtypeStruct((N, D), jnp.float32),
)


def kernel(x, edge_index, W1_out, b1, W1_root, W2_out, b2, W2_root):
    e = edge_index.astype(jnp.int32)
    e = jnp.pad(e, ((0, 0), (0, EPAD - E)))
    e2 = e.reshape(2, EROWS, 128)

    colp, y1s, r1 = _pre(e2, x, W1_out, W1_root)

    rowp3 = e2[0].reshape(NSUB, CHUNKS, CHUNK)
    colp3 = colp.reshape(NSUB, CHUNKS, CHUNK)

    zslab = jnp.zeros((SLAB0, DH), jnp.float32)
    dslab = jnp.zeros((SLAB0, DEGW), jnp.float32)
    ones = jnp.ones((CHUNK, DEGW), jnp.float32)

    zp1, dp = _segsum(y1s, rowp3, colp3, zslab, dslab, ones)
    y2s, r2 = _combine1(zp1, dp, y1s, r1, b1.reshape(1, D), W2_out, W2_root)
    zp2, _ = _segsum(y2s, rowp3, colp3, zslab, dslab, ones)
    return _combine2(zp2, dp, y2s, r2, b2.reshape(1, D))


# speedup vs baseline: 1.4227x; 1.4089x over previous
import functools

import jax
import jax.numpy as jnp
from jax import lax
from jax.experimental import pallas as pl
from jax.experimental.pallas import tpu as pltpu
from jax.experimental.pallas import tpu_sc as plsc

N = 10000
E = 320000
D = 128
DH = 64

NSUB = 16
CHUNK = 128
NBUF = 2
CHUNKS = 158
EPAD = NSUB * CHUNKS * CHUNK
EROWS = EPAD // 128
EVALID_ROWS = E // 128

NOUT = 10240
NACC = 10368
DUMMY = NOUT
SLAB0 = NACC // NSUB
SLAB1 = NOUT // NSUB
DEGW = 8

_sc_mesh = plsc.VectorSubcoreMesh(core_axis_name="c", subcore_axis_name="s")


def _pre_body(e_ref, x_ref, w1o_ref, w1r_ref, colp_ref, ys_ref, r1_ref):
    row = e_ref[0]
    col = e_ref[1]
    ridx = lax.broadcasted_iota(jnp.int32, (EROWS, 128), 0)
    valid = (ridx < EVALID_ROWS) & (row != col)
    colp_ref[...] = jnp.where(valid, col, DUMMY)
    x = x_ref[...]
    dn = (((1,), (1,)), ((), ()))
    y1 = lax.dot_general(x, w1o_ref[...], dn,
                         preferred_element_type=jnp.float32)
    ys_ref[0] = y1[:, :DH]
    ys_ref[1] = y1[:, DH:]
    r1_ref[...] = lax.dot_general(x, w1r_ref[...], dn,
                                  preferred_element_type=jnp.float32)


_pre = pl.pallas_call(
    _pre_body,
    out_shape=(
        jax.ShapeDtypeStruct((EROWS, 128), jnp.int32),
        jax.ShapeDtypeStruct((2, N, DH), jnp.float32),
        jax.ShapeDtypeStruct((N, D), jnp.float32),
    ),
)


def _segsum_body(ys, rowp, colp, zslab, dslab, ones, zout, dout,
                 rowv, colv, rows0, rows1, onesv, zacc, dacc, sem0, sem1):
    c = lax.axis_index("c")
    s = lax.axis_index("s")

    pltpu.sync_copy(zslab, zacc.at[pl.ds(s * SLAB0, SLAB0)])
    pltpu.sync_copy(dslab, dacc.at[pl.ds(s * SLAB0, SLAB0)])
    pltpu.sync_copy(rowp.at[s], rowv)
    pltpu.sync_copy(colp.at[s], colv)
    pltpu.sync_copy(ones, onesv)
    plsc.subcore_barrier()

    yhalf = ys.at[c]

    def gather(j, buf, sem):
        pltpu.async_copy(yhalf.at[rowv.at[j]], buf, sem)

    def gather_wait(j, buf, sem):
        pltpu.make_async_copy(yhalf.at[rowv.at[j]], buf, sem).wait()

    def scatter(j, buf):
        pltpu.sync_copy(buf, zacc.at[colv.at[j]], add=True)

        @pl.when(c == 0)
        def _():
            pltpu.sync_copy(onesv, dacc.at[colv.at[j]], add=True)

    gather(0, rows0, sem0)
    gather(1, rows1, sem1)

    def body(i, carry):
        j = 2 * i
        gather_wait(j, rows0, sem0)
        scatter(j, rows0)

        @pl.when(j + 2 < CHUNKS)
        def _():
            gather(j + 2, rows0, sem0)

        gather_wait(j + 1, rows1, sem1)
        scatter(j + 1, rows1)

        @pl.when(j + 3 < CHUNKS)
        def _():
            gather(j + 3, rows1, sem1)

        return carry

    lax.fori_loop(0, CHUNKS // 2, body, 0)
    plsc.subcore_barrier()

    pltpu.sync_copy(zacc.at[pl.ds(s * SLAB1, SLAB1)],
                    zout.at[c, pl.ds(s * SLAB1, SLAB1)])

    @pl.when(c == 0)
    def _():
        pltpu.sync_copy(dacc.at[pl.ds(s * SLAB1, SLAB1)],
                        dout.at[pl.ds(s * SLAB1, SLAB1)])


_segsum = functools.partial(
    pl.kernel,
    out_type=(
        jax.ShapeDtypeStruct((2, NOUT, DH), jnp.float32),
        jax.ShapeDtypeStruct((NOUT, DEGW), jnp.float32),
    ),
    mesh=_sc_mesh,
    scratch_types=[
        pltpu.VMEM((CHUNKS, CHUNK), jnp.int32),
        pltpu.VMEM((CHUNKS, CHUNK), jnp.int32),
        pltpu.VMEM((CHUNK, DH), jnp.float32),
        pltpu.VMEM((CHUNK, DH), jnp.float32),
        pltpu.VMEM((CHUNK, DEGW), jnp.float32),
        pltpu.VMEM_SHARED((NACC, DH), jnp.float32),
        pltpu.VMEM_SHARED((NACC, DEGW), jnp.float32),
        pltpu.SemaphoreType.DMA,
        pltpu.SemaphoreType.DMA,
    ],
    compiler_params=pltpu.CompilerParams(use_tc_tiling_on_sc=False),
)(_segsum_body)


def _combine1_body(zp, dp, ys, r1, b1, w2o, w2r, y2s_ref, r2_ref):
    z = jnp.concatenate([zp[0, :N, :], zp[1, :N, :]], axis=1)
    y1 = jnp.concatenate([ys[0], ys[1]], axis=1)
    deg = dp[:N, 0:1]
    deginv = 1.0 / (deg + 1.0)
    h = jnp.maximum((z + y1) * deginv + b1[...] + r1[...], 0.0)
    dn = (((1,), (1,)), ((), ()))
    y2 = lax.dot_general(h, w2o[...], dn,
                         preferred_element_type=jnp.float32)
    y2s_ref[0] = y2[:, :DH]
    y2s_ref[1] = y2[:, DH:]
    r2_ref[...] = lax.dot_general(h, w2r[...], dn,
                                  preferred_element_type=jnp.float32)


_combine1 = pl.pallas_call(
    _combine1_body,
    out_shape=(
        jax.ShapeDtypeStruct((2, N, DH), jnp.float32),
        jax.ShapeDtypeStruct((N, D), jnp.float32),
    ),
)


def _combine2_body(zp, dp, y2s, r2, b2, out_ref):
    z = jnp.concatenate([zp[0, :N, :], zp[1, :N, :]], axis=1)
    y2 = jnp.concatenate([y2s[0], y2s[1]], axis=1)
    deg = dp[:N, 0:1]
    deginv = 1.0 / (deg + 1.0)
    out_ref[...] = (z + y2) * deginv + b2[...] + r2[...]


_combine2 = pl.pallas_call(
    _combine2_body,
    out_shape=jax.ShapeDtypeStruct((N, D), jnp.float32),
)


def kernel(x, edge_index, W1_out, b1, W1_root, W2_out, b2, W2_root):
    e = edge_index.astype(jnp.int32)
    e = jnp.pad(e, ((0, 0), (0, EPAD - E)))
    e2 = e.reshape(2, EROWS, 128)

    colp, y1s, r1 = _pre(e2, x, W1_out, W1_root)

    rowp3 = e2[0].reshape(NSUB, CHUNKS, CHUNK)
    colp3 = colp.reshape(NSUB, CHUNKS, CHUNK)

    zslab = jnp.zeros((SLAB0, DH), jnp.float32)
    dslab = jnp.zeros((SLAB0, DEGW), jnp.float32)
    ones = jnp.ones((CHUNK, DEGW), jnp.float32)

    zp1, dp = _segsum(y1s, rowp3, colp3, zslab, dslab, ones)
    y2s, r2 = _combine1(zp1, dp, y1s, r1, b1.reshape(1, D), W2_out, W2_root)
    zp2, _ = _segsum(y2s, rowp3, colp3, zslab, dslab, ones)
    return _combine2(zp2, dp, y2s, r2, b2.reshape(1, D))


# speedup vs baseline: 1.5675x; 1.1018x over previous
import functools

import jax
import jax.numpy as jnp
from jax import lax
from jax.experimental import pallas as pl
from jax.experimental.pallas import tpu as pltpu
from jax.experimental.pallas import tpu_sc as plsc

N = 10000
E = 320000
D = 128
DH = 64

NSUB = 16
CHUNK = 128
NBUF = 2
CHUNKS = 158
EPAD = NSUB * CHUNKS * CHUNK
EROWS = EPAD // 128
EVALID_ROWS = E // 128

NOUT = 10240
NACC = 10368
DUMMY = NOUT
SLAB0 = NACC // NSUB
SLAB1 = NOUT // NSUB
DEGW = 8

_sc_mesh = plsc.VectorSubcoreMesh(core_axis_name="c", subcore_axis_name="s")


def _pre_body(e_ref, x_ref, w1o_ref, w1r_ref, colp_ref, ys_ref, r1_ref):
    row = e_ref[0]
    col = e_ref[1]
    ridx = lax.broadcasted_iota(jnp.int32, (EROWS, 128), 0)
    valid = (ridx < EVALID_ROWS) & (row != col)
    colp_ref[...] = jnp.where(valid, col, DUMMY)
    x = x_ref[...]
    dn = (((1,), (1,)), ((), ()))
    y1 = lax.dot_general(x, w1o_ref[...], dn,
                         preferred_element_type=jnp.float32)
    ys_ref[0] = y1[:, :DH]
    ys_ref[1] = y1[:, DH:]
    r1_ref[...] = lax.dot_general(x, w1r_ref[...], dn,
                                  preferred_element_type=jnp.float32)


_pre = pl.pallas_call(
    _pre_body,
    out_shape=(
        jax.ShapeDtypeStruct((EROWS, 128), jnp.int32),
        jax.ShapeDtypeStruct((2, N, DH), jnp.float32),
        jax.ShapeDtypeStruct((N, D), jnp.float32),
    ),
)


def _segsum_body(ys, rowp, colp, zslab, dslab, ones, zout, dout,
                 rowv, colv, rows0, rows1, rows2, rows3, onesv, zacc, dacc,
                 sem0, sem1, sem2, sem3):
    c = lax.axis_index("c")
    s = lax.axis_index("s")

    pltpu.sync_copy(zslab, zacc.at[pl.ds(s * SLAB0, SLAB0)])
    pltpu.sync_copy(dslab, dacc.at[pl.ds(s * SLAB0, SLAB0)])
    pltpu.sync_copy(rowp.at[s], rowv)
    pltpu.sync_copy(colp.at[s], colv)
    pltpu.sync_copy(ones, onesv)
    plsc.subcore_barrier()

    yhalf = ys.at[c]

    def gather(j, buf, sem):
        pltpu.async_copy(yhalf.at[rowv.at[j]], buf, sem)

    def gather_wait(j, buf, sem):
        pltpu.make_async_copy(yhalf.at[rowv.at[j]], buf, sem).wait()

    def scatter(j, buf):
        pltpu.sync_copy(buf, zacc.at[colv.at[j]], add=True)

        @pl.when(c == 0)
        def _():
            pltpu.sync_copy(onesv, dacc.at[colv.at[j]], add=True)

    bufs = ((rows0, sem0), (rows1, sem1), (rows2, sem2), (rows3, sem3))
    for b, (buf, sem) in enumerate(bufs):
        gather(b, buf, sem)

    NB = len(bufs)
    NITER = (CHUNKS + NB - 1) // NB

    def body(i, carry):
        j = NB * i
        for b, (buf, sem) in enumerate(bufs):
            @pl.when(j + b < CHUNKS)
            def _():
                gather_wait(j + b, buf, sem)
                scatter(j + b, buf)

            @pl.when(j + b + NB < CHUNKS)
            def _():
                gather(j + b + NB, buf, sem)

        return carry

    lax.fori_loop(0, NITER, body, 0)
    plsc.subcore_barrier()

    pltpu.sync_copy(zacc.at[pl.ds(s * SLAB1, SLAB1)],
                    zout.at[c, pl.ds(s * SLAB1, SLAB1)])

    @pl.when(c == 0)
    def _():
        pltpu.sync_copy(dacc.at[pl.ds(s * SLAB1, SLAB1)],
                        dout.at[pl.ds(s * SLAB1, SLAB1)])


_segsum = functools.partial(
    pl.kernel,
    out_type=(
        jax.ShapeDtypeStruct((2, NOUT, DH), jnp.float32),
        jax.ShapeDtypeStruct((NOUT, DEGW), jnp.float32),
    ),
    mesh=_sc_mesh,
    scratch_types=[
        pltpu.VMEM((CHUNKS, CHUNK), jnp.int32),
        pltpu.VMEM((CHUNKS, CHUNK), jnp.int32),
        pltpu.VMEM((CHUNK, DH), jnp.float32),
        pltpu.VMEM((CHUNK, DH), jnp.float32),
        pltpu.VMEM((CHUNK, DH), jnp.float32),
        pltpu.VMEM((CHUNK, DH), jnp.float32),
        pltpu.VMEM((CHUNK, DEGW), jnp.float32),
        pltpu.VMEM_SHARED((NACC, DH), jnp.float32),
        pltpu.VMEM_SHARED((NACC, DEGW), jnp.float32),
        pltpu.SemaphoreType.DMA,
        pltpu.SemaphoreType.DMA,
        pltpu.SemaphoreType.DMA,
        pltpu.SemaphoreType.DMA,
    ],
    compiler_params=pltpu.CompilerParams(use_tc_tiling_on_sc=False),
)(_segsum_body)


def _combine1_body(zp, dp, ys, r1, b1, w2o, w2r, y2s_ref, r2_ref):
    z = jnp.concatenate([zp[0, :N, :], zp[1, :N, :]], axis=1)
    y1 = jnp.concatenate([ys[0], ys[1]], axis=1)
    deg = dp[:N, 0:1]
    deginv = 1.0 / (deg + 1.0)
    h = jnp.maximum((z + y1) * deginv + b1[...] + r1[...], 0.0)
    dn = (((1,), (1,)), ((), ()))
    y2 = lax.dot_general(h, w2o[...], dn,
                         preferred_element_type=jnp.float32)
    y2s_ref[0] = y2[:, :DH]
    y2s_ref[1] = y2[:, DH:]
    r2_ref[...] = lax.dot_general(h, w2r[...], dn,
                                  preferred_element_type=jnp.float32)


_combine1 = pl.pallas_call(
    _combine1_body,
    out_shape=(
        jax.ShapeDtypeStruct((2, N, DH), jnp.float32),
        jax.ShapeDtypeStruct((N, D), jnp.float32),
    ),
)


def _combine2_body(zp, dp, y2s, r2, b2, out_ref):
    z = jnp.concatenate([zp[0, :N, :], zp[1, :N, :]], axis=1)
    y2 = jnp.concatenate([y2s[0], y2s[1]], axis=1)
    deg = dp[:N, 0:1]
    deginv = 1.0 / (deg + 1.0)
    out_ref[...] = (z + y2) * deginv + b2[...] + r2[...]


_combine2 = pl.pallas_call(
    _combine2_body,
    out_shape=jax.ShapeDtypeStruct((N, D), jnp.float32),
)


def kernel(x, edge_index, W1_out, b1, W1_root, W2_out, b2, W2_root):
    e = edge_index.astype(jnp.int32)
    e = jnp.pad(e, ((0, 0), (0, EPAD - E)))
    e2 = e.reshape(2, EROWS, 128)

    colp, y1s, r1 = _pre(e2, x, W1_out, W1_root)

    rowp3 = e2[0].reshape(NSUB, CHUNKS, CHUNK)
    colp3 = colp.reshape(NSUB, CHUNKS, CHUNK)

    zslab = jnp.zeros((SLAB0, DH), jnp.float32)
    dslab = jnp.zeros((SLAB0, DEGW), jnp.float32)
    ones = jnp.ones((CHUNK, DEGW), jnp.float32)

    zp1, dp = _segsum(y1s, rowp3, colp3, zslab, dslab, ones)
    y2s, r2 = _combine1(zp1, dp, y1s, r1, b1.reshape(1, D), W2_out, W2_root)
    zp2, _ = _segsum(y2s, rowp3, colp3, zslab, dslab, ones)
    return _combine2(zp2, dp, y2s, r2, b2.reshape(1, D))


# speedup vs baseline: 1.6113x; 1.0279x over previous
import functools

import jax
import jax.numpy as jnp
from jax import lax
from jax.experimental import pallas as pl
from jax.experimental.pallas import tpu as pltpu
from jax.experimental.pallas import tpu_sc as plsc

N = 10000
E = 320000
D = 128
DH = 64

NSUB = 16
CHUNK = 128
NBUF = 2
CHUNKS = 158
EPAD = NSUB * CHUNKS * CHUNK
EROWS = EPAD // 128
EVALID_ROWS = E // 128

NOUT = 10240
NACC = 10368
DUMMY = NOUT
SLAB0 = NACC // NSUB
SLAB1 = NOUT // NSUB
DEGW = 8

_sc_mesh = plsc.VectorSubcoreMesh(core_axis_name="c", subcore_axis_name="s")


def _pre_body(e_ref, x_ref, w1o_ref, w1r_ref, colp_ref, ys_ref, r1_ref):
    row = e_ref[0]
    col = e_ref[1]
    ridx = lax.broadcasted_iota(jnp.int32, (EROWS, 128), 0)
    valid = (ridx < EVALID_ROWS) & (row != col)
    colp_ref[...] = jnp.where(valid, col, DUMMY)
    x = x_ref[...]
    dn = (((1,), (1,)), ((), ()))
    y1 = lax.dot_general(x, w1o_ref[...], dn,
                         preferred_element_type=jnp.float32)
    ys_ref[0] = y1[:, :DH]
    ys_ref[1] = y1[:, DH:]
    r1_ref[...] = lax.dot_general(x, w1r_ref[...], dn,
                                  preferred_element_type=jnp.float32)


_pre = pl.pallas_call(
    _pre_body,
    out_shape=(
        jax.ShapeDtypeStruct((EROWS, 128), jnp.int32),
        jax.ShapeDtypeStruct((2, N, DH), jnp.float32),
        jax.ShapeDtypeStruct((N, D), jnp.float32),
    ),
)


def _segsum_pipeline(c, s, ys, rowp, colp, zslab, zout, rowv, colv, bufs,
                     zacc, deg):
    pltpu.sync_copy(zslab, zacc.at[pl.ds(s * SLAB0, SLAB0)])
    pltpu.sync_copy(rowp.at[s], rowv)
    pltpu.sync_copy(colp.at[s], colv)
    if deg is not None:
        dslab, ones, onesv, dacc, dout = deg
        pltpu.sync_copy(dslab, dacc.at[pl.ds(s * SLAB0, SLAB0)])
        pltpu.sync_copy(ones, onesv)
    plsc.subcore_barrier()

    yhalf = ys.at[c]

    def gather(j, buf, sem):
        pltpu.async_copy(yhalf.at[rowv.at[j]], buf, sem)

    def gather_wait(j, buf, sem):
        pltpu.make_async_copy(yhalf.at[rowv.at[j]], buf, sem).wait()

    def scatter(j, buf):
        pltpu.sync_copy(buf, zacc.at[colv.at[j]], add=True)

        if deg is not None:
            @pl.when(c == 0)
            def _():
                pltpu.sync_copy(onesv, dacc.at[colv.at[j]], add=True)

    for b, (buf, sem) in enumerate(bufs):
        gather(b, buf, sem)

    NB = len(bufs)
    NITER = (CHUNKS + NB - 1) // NB

    def body(i, carry):
        j = NB * i
        for b, (buf, sem) in enumerate(bufs):
            @pl.when(j + b < CHUNKS)
            def _():
                gather_wait(j + b, buf, sem)
                scatter(j + b, buf)

            @pl.when(j + b + NB < CHUNKS)
            def _():
                gather(j + b + NB, buf, sem)

        return carry

    lax.fori_loop(0, NITER, body, 0)
    plsc.subcore_barrier()

    pltpu.sync_copy(zacc.at[pl.ds(s * SLAB1, SLAB1)],
                    zout.at[c, pl.ds(s * SLAB1, SLAB1)])

    if deg is not None:
        @pl.when(c == 0)
        def _():
            pltpu.sync_copy(dacc.at[pl.ds(s * SLAB1, SLAB1)],
                            dout.at[pl.ds(s * SLAB1, SLAB1)])


def _segsum_deg_body(ys, rowp, colp, zslab, dslab, ones, zout, dout,
                     rowv, colv, rows0, rows1, rows2, rows3, rows4,
                     onesv, zacc, dacc,
                     sem0, sem1, sem2, sem3, sem4):
    c = lax.axis_index("c")
    s = lax.axis_index("s")
    bufs = ((rows0, sem0), (rows1, sem1), (rows2, sem2), (rows3, sem3),
            (rows4, sem4))
    _segsum_pipeline(c, s, ys, rowp, colp, zslab, zout, rowv, colv, bufs,
                     zacc, (dslab, ones, onesv, dacc, dout))


def _segsum_nodeg_body(ys, rowp, colp, zslab, zout,
                       rowv, colv, rows0, rows1, rows2, rows3, rows4,
                       zacc, sem0, sem1, sem2, sem3, sem4):
    c = lax.axis_index("c")
    s = lax.axis_index("s")
    bufs = ((rows0, sem0), (rows1, sem1), (rows2, sem2), (rows3, sem3),
            (rows4, sem4))
    _segsum_pipeline(c, s, ys, rowp, colp, zslab, zout, rowv, colv, bufs,
                     zacc, None)


_PIPE_SCRATCH = [
    pltpu.VMEM((CHUNKS, CHUNK), jnp.int32),
    pltpu.VMEM((CHUNKS, CHUNK), jnp.int32),
    pltpu.VMEM((CHUNK, DH), jnp.float32),
    pltpu.VMEM((CHUNK, DH), jnp.float32),
    pltpu.VMEM((CHUNK, DH), jnp.float32),
    pltpu.VMEM((CHUNK, DH), jnp.float32),
    pltpu.VMEM((CHUNK, DH), jnp.float32),
]
_PIPE_SEMS = [pltpu.SemaphoreType.DMA] * 5

_segsum_deg = functools.partial(
    pl.kernel,
    out_type=(
        jax.ShapeDtypeStruct((2, NOUT, DH), jnp.float32),
        jax.ShapeDtypeStruct((NOUT, DEGW), jnp.float32),
    ),
    mesh=_sc_mesh,
    scratch_types=_PIPE_SCRATCH + [
        pltpu.VMEM((CHUNK, DEGW), jnp.float32),
        pltpu.VMEM_SHARED((NACC, DH), jnp.float32),
        pltpu.VMEM_SHARED((NACC, DEGW), jnp.float32),
    ] + _PIPE_SEMS,
    compiler_params=pltpu.CompilerParams(use_tc_tiling_on_sc=False),
)(_segsum_deg_body)

_segsum_nodeg = functools.partial(
    pl.kernel,
    out_type=jax.ShapeDtypeStruct((2, NOUT, DH), jnp.float32),
    mesh=_sc_mesh,
    scratch_types=_PIPE_SCRATCH + [
        pltpu.VMEM_SHARED((NACC, DH), jnp.float32),
    ] + _PIPE_SEMS,
    compiler_params=pltpu.CompilerParams(use_tc_tiling_on_sc=False),
)(_segsum_nodeg_body)


def _combine1_body(zp, dp, ys, r1, b1, w2o, w2r, y2s_ref, r2_ref):
    z = jnp.concatenate([zp[0, :N, :], zp[1, :N, :]], axis=1)
    y1 = jnp.concatenate([ys[0], ys[1]], axis=1)
    deg = dp[:N, 0:1]
    deginv = 1.0 / (deg + 1.0)
    h = jnp.maximum((z + y1) * deginv + b1[...] + r1[...], 0.0)
    dn = (((1,), (1,)), ((), ()))
    y2 = lax.dot_general(h, w2o[...], dn,
                         preferred_element_type=jnp.float32)
    y2s_ref[0] = y2[:, :DH]
    y2s_ref[1] = y2[:, DH:]
    r2_ref[...] = lax.dot_general(h, w2r[...], dn,
                                  preferred_element_type=jnp.float32)


_combine1 = pl.pallas_call(
    _combine1_body,
    out_shape=(
        jax.ShapeDtypeStruct((2, N, DH), jnp.float32),
        jax.ShapeDtypeStruct((N, D), jnp.float32),
    ),
)


def _combine2_body(zp, dp, y2s, r2, b2, out_ref):
    z = jnp.concatenate([zp[0, :N, :], zp[1, :N, :]], axis=1)
    y2 = jnp.concatenate([y2s[0], y2s[1]], axis=1)
    deg = dp[:N, 0:1]
    deginv = 1.0 / (deg + 1.0)
    out_ref[...] = (z + y2) * deginv + b2[...] + r2[...]


_combine2 = pl.pallas_call(
    _combine2_body,
    out_shape=jax.ShapeDtypeStruct((N, D), jnp.float32),
)


def kernel(x, edge_index, W1_out, b1, W1_root, W2_out, b2, W2_root):
    e = edge_index.astype(jnp.int32)
    e = jnp.pad(e, ((0, 0), (0, EPAD - E)))
    e2 = e.reshape(2, EROWS, 128)

    colp, y1s, r1 = _pre(e2, x, W1_out, W1_root)

    rowp3 = e2[0].reshape(NSUB, CHUNKS, CHUNK)
    colp3 = colp.reshape(NSUB, CHUNKS, CHUNK)

    zslab = jnp.zeros((SLAB0, DH), jnp.float32)
    dslab = jnp.zeros((SLAB0, DEGW), jnp.float32)
    ones = jnp.ones((CHUNK, DEGW), jnp.float32)

    zp1, dp = _segsum_deg(y1s, rowp3, colp3, zslab, dslab, ones)
    y2s, r2 = _combine1(zp1, dp, y1s, r1, b1.reshape(1, D), W2_out, W2_root)
    zp2 = _segsum_nodeg(y2s, rowp3, colp3, zslab)
    return _combine2(zp2, dp, y2s, r2, b2.reshape(1, D))


# speedup vs baseline: 1.6184x; 1.0044x over previous
import functools

import jax
import jax.numpy as jnp
from jax import lax
from jax.experimental import pallas as pl
from jax.experimental.pallas import tpu as pltpu
from jax.experimental.pallas import tpu_sc as plsc

N = 10000
E = 320000
D = 128
DH = 64

NSUB = 16
CHUNK = 128
NBUF = 2
CHUNKS = 158
EPAD = NSUB * CHUNKS * CHUNK
EROWS = EPAD // 128
EVALID_ROWS = E // 128

NOUT = 10240
NACC = 10368
DUMMY = NOUT
SLAB0 = NACC // NSUB
SLAB1 = NOUT // NSUB
DEGW = 8

_sc_mesh = plsc.VectorSubcoreMesh(core_axis_name="c", subcore_axis_name="s")


def _pre_body(e_ref, x_ref, w1o_ref, w1r_ref, colp_ref, ys_ref, r1_ref):
    row = e_ref[0]
    col = e_ref[1]
    ridx = lax.broadcasted_iota(jnp.int32, (EROWS, 128), 0)
    valid = (ridx < EVALID_ROWS) & (row != col)
    colp_ref[...] = jnp.where(valid, col, DUMMY)
    x = x_ref[...]
    dn = (((1,), (1,)), ((), ()))
    y1 = lax.dot_general(x, w1o_ref[...], dn,
                         preferred_element_type=jnp.float32)
    ys_ref[0] = y1[:, :DH]
    ys_ref[1] = y1[:, DH:]
    r1_ref[...] = lax.dot_general(x, w1r_ref[...], dn,
                                  preferred_element_type=jnp.float32)


_pre = pl.pallas_call(
    _pre_body,
    out_shape=(
        jax.ShapeDtypeStruct((EROWS, 128), jnp.int32),
        jax.ShapeDtypeStruct((2, N, DH), jnp.float32),
        jax.ShapeDtypeStruct((N, D), jnp.float32),
    ),
)


def _segsum_pipeline(c, s, ys, rowp, colp, zslab, zout, rowv, colv, bufs,
                     zacc, deg):
    z0 = jnp.minimum(s * SLAB0, NACC - SLAB0)
    pltpu.sync_copy(zslab, zacc.at[pl.ds(z0, SLAB0)])
    pltpu.sync_copy(rowp.at[s], rowv)
    pltpu.sync_copy(colp.at[s], colv)
    if deg is not None:
        dslab, ones, onesv, dacc, dout = deg
        pltpu.sync_copy(dslab, dacc.at[pl.ds(z0, SLAB0)])
        pltpu.sync_copy(ones, onesv)
    plsc.subcore_barrier()

    yhalf = ys.at[c]

    def gather(j, buf, sem):
        pltpu.async_copy(yhalf.at[rowv.at[j]], buf, sem)

    def gather_wait(j, buf, sem):
        pltpu.make_async_copy(yhalf.at[rowv.at[j]], buf, sem).wait()

    def scatter(j, buf):
        pltpu.sync_copy(buf, zacc.at[colv.at[j]], add=True)

        if deg is not None:
            @pl.when(c == 0)
            def _():
                pltpu.sync_copy(onesv, dacc.at[colv.at[j]], add=True)

    for b, (buf, sem) in enumerate(bufs):
        gather(b, buf, sem)

    NB = len(bufs)
    NITER = (CHUNKS + NB - 1) // NB

    def body(i, carry):
        j = NB * i
        for b, (buf, sem) in enumerate(bufs):
            @pl.when(j + b < CHUNKS)
            def _():
                gather_wait(j + b, buf, sem)
                scatter(j + b, buf)

            @pl.when(j + b + NB < CHUNKS)
            def _():
                gather(j + b + NB, buf, sem)

        return carry

    lax.fori_loop(0, NITER, body, 0)
    plsc.subcore_barrier()

    pltpu.sync_copy(zacc.at[pl.ds(s * SLAB1, SLAB1)],
                    zout.at[c, pl.ds(s * SLAB1, SLAB1)])

    if deg is not None:
        @pl.when(c == 0)
        def _():
            pltpu.sync_copy(dacc.at[pl.ds(s * SLAB1, SLAB1)],
                            dout.at[pl.ds(s * SLAB1, SLAB1)])


def _segsum_deg_body(ys, rowp, colp, zslab, dslab, ones, zout, dout,
                     rowv, colv, rows0, rows1, rows2, rows3, rows4,
                     onesv, zacc, dacc,
                     sem0, sem1, sem2, sem3, sem4):
    c = lax.axis_index("c")
    s = lax.axis_index("s")
    bufs = ((rows0, sem0), (rows1, sem1), (rows2, sem2), (rows3, sem3),
            (rows4, sem4))
    _segsum_pipeline(c, s, ys, rowp, colp, zslab, zout, rowv, colv, bufs,
                     zacc, (dslab, ones, onesv, dacc, dout))


def _segsum_nodeg_body(ys, rowp, colp, zslab, zout,
                       rowv, colv, rows0, rows1, rows2, rows3, rows4,
                       zacc, sem0, sem1, sem2, sem3, sem4):
    c = lax.axis_index("c")
    s = lax.axis_index("s")
    bufs = ((rows0, sem0), (rows1, sem1), (rows2, sem2), (rows3, sem3),
            (rows4, sem4))
    _segsum_pipeline(c, s, ys, rowp, colp, zslab, zout, rowv, colv, bufs,
                     zacc, None)


_PIPE_SCRATCH = [
    pltpu.VMEM((CHUNKS, CHUNK), jnp.int32),
    pltpu.VMEM((CHUNKS, CHUNK), jnp.int32),
    pltpu.VMEM((CHUNK, DH), jnp.float32),
    pltpu.VMEM((CHUNK, DH), jnp.float32),
    pltpu.VMEM((CHUNK, DH), jnp.float32),
    pltpu.VMEM((CHUNK, DH), jnp.float32),
    pltpu.VMEM((CHUNK, DH), jnp.float32),
]
_PIPE_SEMS = [pltpu.SemaphoreType.DMA] * 5

_segsum_deg = functools.partial(
    pl.kernel,
    out_type=(
        jax.ShapeDtypeStruct((2, NOUT, DH), jnp.float32),
        jax.ShapeDtypeStruct((NOUT, DEGW), jnp.float32),
    ),
    mesh=_sc_mesh,
    scratch_types=_PIPE_SCRATCH + [
        pltpu.VMEM((CHUNK, DEGW), jnp.float32),
        pltpu.VMEM_SHARED((NACC, DH), jnp.float32),
        pltpu.VMEM_SHARED((NACC, DEGW), jnp.float32),
    ] + _PIPE_SEMS,
    compiler_params=pltpu.CompilerParams(use_tc_tiling_on_sc=False),
)(_segsum_deg_body)

_segsum_nodeg = functools.partial(
    pl.kernel,
    out_type=jax.ShapeDtypeStruct((2, NOUT, DH), jnp.float32),
    mesh=_sc_mesh,
    scratch_types=_PIPE_SCRATCH + [
        pltpu.VMEM_SHARED((NACC, DH), jnp.float32),
    ] + _PIPE_SEMS,
    compiler_params=pltpu.CompilerParams(use_tc_tiling_on_sc=False),
)(_segsum_nodeg_body)


def _combine1_body(zp, dp, ys, r1, b1, w2o, w2r, y2s_ref, r2_ref):
    z = jnp.concatenate([zp[0, :N, :], zp[1, :N, :]], axis=1)
    y1 = jnp.concatenate([ys[0], ys[1]], axis=1)
    deg = dp[:N, 0:1]
    deginv = 1.0 / (deg + 1.0)
    h = jnp.maximum((z + y1) * deginv + b1[...] + r1[...], 0.0)
    dn = (((1,), (1,)), ((), ()))
    y2 = lax.dot_general(h, w2o[...], dn,
                         preferred_element_type=jnp.float32)
    y2s_ref[0] = y2[:, :DH]
    y2s_ref[1] = y2[:, DH:]
    r2_ref[...] = lax.dot_general(h, w2r[...], dn,
                                  preferred_element_type=jnp.float32)


_combine1 = pl.pallas_call(
    _combine1_body,
    out_shape=(
        jax.ShapeDtypeStruct((2, N, DH), jnp.float32),
        jax.ShapeDtypeStruct((N, D), jnp.float32),
    ),
)


def _combine2_body(zp, dp, y2s, r2, b2, out_ref):
    z = jnp.concatenate([zp[0, :N, :], zp[1, :N, :]], axis=1)
    y2 = jnp.concatenate([y2s[0], y2s[1]], axis=1)
    deg = dp[:N, 0:1]
    deginv = 1.0 / (deg + 1.0)
    out_ref[...] = (z + y2) * deginv + b2[...] + r2[...]


_combine2 = pl.pallas_call(
    _combine2_body,
    out_shape=jax.ShapeDtypeStruct((N, D), jnp.float32),
)


def kernel(x, edge_index, W1_out, b1, W1_root, W2_out, b2, W2_root):
    e = edge_index.astype(jnp.int32)
    e = jnp.pad(e, ((0, 0), (0, EPAD - E)))
    e2 = e.reshape(2, EROWS, 128)

    colp, y1s, r1 = _pre(e2, x, W1_out, W1_root)

    rowp3 = e2[0].reshape(NSUB, CHUNKS, CHUNK)
    colp3 = colp.reshape(NSUB, CHUNKS, CHUNK)

    zslab = jnp.zeros((SLAB0, DH), jnp.float32)
    dslab = jnp.zeros((SLAB0, DEGW), jnp.float32)
    ones = jnp.ones((CHUNK, DEGW), jnp.float32)

    zp1, dp = _segsum_deg(y1s, rowp3, colp3, zslab, dslab, ones)
    y2s, r2 = _combine1(zp1, dp, y1s, r1, b1.reshape(1, D), W2_out, W2_root)
    zp2 = _segsum_nodeg(y2s, rowp3, colp3, zslab)
    return _combine2(zp2, dp, y2s, r2, b2.reshape(1, D))


# speedup vs baseline: 1.6395x; 1.0130x over previous
import functools

import jax
import jax.numpy as jnp
from jax import lax
from jax.experimental import pallas as pl
from jax.experimental.pallas import tpu as pltpu
from jax.experimental.pallas import tpu_sc as plsc

N = 10000
E = 320000
D = 128
DH = 64

NSUB = 16
CHUNK = 128
CHUNKS = 158
EPAD = NSUB * CHUNKS * CHUNK
EROWS = EPAD // 128
EVALID_ROWS = E // 128

NOUT = 10240
NACC = 10368
DUMMY = NOUT
SLAB0 = NACC // NSUB
SLAB1 = NOUT // NSUB
DEGW = 8

_sc_mesh = plsc.VectorSubcoreMesh(core_axis_name="c", subcore_axis_name="s")


def _pre_body(e_ref, x_ref, w1o_ref, w1r_ref, colp_ref, ys_ref, r1_ref):
    row = e_ref[0]
    col = e_ref[1]
    ridx = lax.broadcasted_iota(jnp.int32, (EROWS, 128), 0)
    valid = (ridx < EVALID_ROWS) & (row != col)
    colp_ref[...] = jnp.where(valid, col, DUMMY)
    x = x_ref[...]
    dn = (((1,), (1,)), ((), ()))
    y1 = lax.dot_general(x, w1o_ref[...], dn,
                         preferred_element_type=jnp.float32)
    ys_ref[0] = y1[:, :DH]
    ys_ref[1] = y1[:, DH:]
    r1_ref[...] = lax.dot_general(x, w1r_ref[...], dn,
                                  preferred_element_type=jnp.float32)


_pre = pl.pallas_call(
    _pre_body,
    out_shape=(
        jax.ShapeDtypeStruct((EROWS, 128), jnp.int32),
        jax.ShapeDtypeStruct((2, N, DH), jnp.float32),
        jax.ShapeDtypeStruct((N, D), jnp.float32),
    ),
)


def _segsum_pipeline(c, s, ys, rowp, colp, zslab, zout, rowv, colv, bufs,
                     zacc, deg):
    z0 = jnp.minimum(s * SLAB0, NACC - SLAB0)
    pltpu.sync_copy(zslab, zacc.at[pl.ds(z0, SLAB0)])
    pltpu.sync_copy(rowp.at[s], rowv)
    pltpu.sync_copy(colp.at[s], colv)
    if deg is not None:
        dslab, ones, onesv, dacc, dout = deg
        pltpu.sync_copy(dslab, dacc.at[pl.ds(z0, SLAB0)])
        pltpu.sync_copy(ones, onesv)
    plsc.subcore_barrier()

    yhalf = ys.at[c]

    def gather(j, buf, sem):
        pltpu.async_copy(yhalf.at[rowv.at[j]], buf, sem)

    def gather_wait(j, buf, sem):
        pltpu.make_async_copy(yhalf.at[rowv.at[j]], buf, sem).wait()

    def scatter(j, buf):
        pltpu.sync_copy(buf, zacc.at[colv.at[j]], add=True)

        if deg is not None:
            @pl.when(c == 0)
            def _():
                pltpu.sync_copy(onesv, dacc.at[colv.at[j]], add=True)

    for b, (buf, sem) in enumerate(bufs):
        gather(b, buf, sem)

    NB = len(bufs)
    NITER = (CHUNKS + NB - 1) // NB

    def body(i, carry):
        j = NB * i
        for b, (buf, sem) in enumerate(bufs):
            @pl.when(j + b < CHUNKS)
            def _():
                gather_wait(j + b, buf, sem)
                scatter(j + b, buf)

            @pl.when(j + b + NB < CHUNKS)
            def _():
                gather(j + b + NB, buf, sem)

        return carry

    lax.fori_loop(0, NITER, body, 0)
    plsc.subcore_barrier()

    pltpu.sync_copy(zacc.at[pl.ds(s * SLAB1, SLAB1)],
                    zout.at[c, pl.ds(s * SLAB1, SLAB1)])

    if deg is not None:
        @pl.when(c == 0)
        def _():
            pltpu.sync_copy(dacc.at[pl.ds(s * SLAB1, SLAB1)],
                            dout.at[pl.ds(s * SLAB1, SLAB1)])


def _segsum_deg_body(ys, rowp, colp, zslab, dslab, ones, zout, dout,
                     rowv, colv, rows0, rows1, rows2, rows3, rows4,
                     onesv, zacc, dacc,
                     sem0, sem1, sem2, sem3, sem4):
    c = lax.axis_index("c")
    s = lax.axis_index("s")
    bufs = ((rows0, sem0), (rows1, sem1), (rows2, sem2), (rows3, sem3),
            (rows4, sem4))
    _segsum_pipeline(c, s, ys, rowp, colp, zslab, zout, rowv, colv, bufs,
                     zacc, (dslab, ones, onesv, dacc, dout))


def _segsum_nodeg_body(ys, rowp, colp, zslab, zout,
                       rowv, colv, rows0, rows1, rows2, rows3, rows4,
                       zacc, sem0, sem1, sem2, sem3, sem4):
    c = lax.axis_index("c")
    s = lax.axis_index("s")
    bufs = ((rows0, sem0), (rows1, sem1), (rows2, sem2), (rows3, sem3),
            (rows4, sem4))
    _segsum_pipeline(c, s, ys, rowp, colp, zslab, zout, rowv, colv, bufs,
                     zacc, None)


_PIPE_SCRATCH = [
    pltpu.VMEM((CHUNKS, CHUNK), jnp.int32),
    pltpu.VMEM((CHUNKS, CHUNK), jnp.int32),
    pltpu.VMEM((CHUNK, DH), jnp.float32),
    pltpu.VMEM((CHUNK, DH), jnp.float32),
    pltpu.VMEM((CHUNK, DH), jnp.float32),
    pltpu.VMEM((CHUNK, DH), jnp.float32),
    pltpu.VMEM((CHUNK, DH), jnp.float32),
]
_PIPE_SEMS = [pltpu.SemaphoreType.DMA] * 5

_segsum_deg = functools.partial(
    pl.kernel,
    out_type=(
        jax.ShapeDtypeStruct((2, NOUT, DH), jnp.float32),
        jax.ShapeDtypeStruct((NOUT, DEGW), jnp.float32),
    ),
    mesh=_sc_mesh,
    scratch_types=_PIPE_SCRATCH + [
        pltpu.VMEM((CHUNK, DEGW), jnp.float32),
        pltpu.VMEM_SHARED((NACC, DH), jnp.float32),
        pltpu.VMEM_SHARED((NACC, DEGW), jnp.float32),
    ] + _PIPE_SEMS,
    compiler_params=pltpu.CompilerParams(use_tc_tiling_on_sc=False),
)(_segsum_deg_body)

_segsum_nodeg = functools.partial(
    pl.kernel,
    out_type=jax.ShapeDtypeStruct((2, NOUT, DH), jnp.float32),
    mesh=_sc_mesh,
    scratch_types=_PIPE_SCRATCH + [
        pltpu.VMEM_SHARED((NACC, DH), jnp.float32),
    ] + _PIPE_SEMS,
    compiler_params=pltpu.CompilerParams(use_tc_tiling_on_sc=False),
)(_segsum_nodeg_body)


def _combine1_body(zp, dp, ys, r1, b1, w2o, w2r, y2s_ref, r2_ref):
    z = jnp.concatenate([zp[0, :N, :], zp[1, :N, :]], axis=1)
    y1 = jnp.concatenate([ys[0], ys[1]], axis=1)
    deg = dp[:N, 0:1]
    deginv = 1.0 / (deg + 1.0)
    h = jnp.maximum((z + y1) * deginv + b1[...] + r1[...], 0.0)
    dn = (((1,), (1,)), ((), ()))
    y2 = lax.dot_general(h, w2o[...], dn,
                         preferred_element_type=jnp.float32)
    y2s_ref[0] = y2[:, :DH]
    y2s_ref[1] = y2[:, DH:]
    r2_ref[...] = lax.dot_general(h, w2r[...], dn,
                                  preferred_element_type=jnp.float32)


_combine1 = pl.pallas_call(
    _combine1_body,
    out_shape=(
        jax.ShapeDtypeStruct((2, N, DH), jnp.float32),
        jax.ShapeDtypeStruct((N, D), jnp.float32),
    ),
)


def _combine2_body(zp, dp, y2s, r2, b2, out_ref):
    z = jnp.concatenate([zp[0, :N, :], zp[1, :N, :]], axis=1)
    y2 = jnp.concatenate([y2s[0], y2s[1]], axis=1)
    deg = dp[:N, 0:1]
    deginv = 1.0 / (deg + 1.0)
    out_ref[...] = (z + y2) * deginv + b2[...] + r2[...]


_combine2 = pl.pallas_call(
    _combine2_body,
    out_shape=jax.ShapeDtypeStruct((N, D), jnp.float32),
)


def kernel(x, edge_index, W1_out, b1, W1_root, W2_out, b2, W2_root):
    e = edge_index.astype(jnp.int32)
    e = jnp.pad(e, ((0, 0), (0, EPAD - E)))
    e2 = e.reshape(2, EROWS, 128)

    colp, y1s, r1 = _pre(e2, x, W1_out, W1_root)

    rowp3 = e2[0].reshape(NSUB, CHUNKS, CHUNK)
    colp3 = colp.reshape(NSUB, CHUNKS, CHUNK)

    zslab = jnp.zeros((SLAB0, DH), jnp.float32)
    dslab = jnp.zeros((SLAB0, DEGW), jnp.float32)
    ones = jnp.ones((CHUNK, DEGW), jnp.float32)

    zp1, dp = _segsum_deg(y1s, rowp3, colp3, zslab, dslab, ones)
    y2s, r2 = _combine1(zp1, dp, y1s, r1, b1.reshape(1, D), W2_out, W2_root)
    zp2 = _segsum_nodeg(y2s, rowp3, colp3, zslab)
    return _combine2(zp2, dp, y2s, r2, b2.reshape(1, D))
